# baseline (speedup 1.0000x reference)
import functools

import jax
import jax.numpy as jnp
from jax import lax
from jax.experimental import pallas as pl
from jax.experimental.pallas import tpu as pltpu
from jax.experimental.pallas import tpu_sc as plsc

NC = 2
NS = 16
NW = NC * NS
UNROLL = 16


CHUNK = 32


def _gather_body(bpw, u_tab, m_tab, uidx_hbm, midx_hbm, u_out, m_out,
                 uidx_v, midx_v, rows_v, sem, sem2):
    wid = lax.axis_index("s") * NC + lax.axis_index("c")
    base = wid * bpw
    nchunk = bpw // CHUNK
    pltpu.sync_copy(uidx_hbm.at[pl.ds(base, bpw)], uidx_v)
    pltpu.sync_copy(midx_hbm.at[pl.ds(base, bpw)], midx_v)
    for idx_v, tab, out, s in ((uidx_v, u_tab, u_out, sem),
                               (midx_v, m_tab, m_out, sem2)):
        for j in range(nchunk):
            pltpu.async_copy(tab.at[idx_v.at[pl.ds(j * CHUNK, CHUNK)]],
                             rows_v.at[pl.ds(j * CHUNK, CHUNK)], s)
        pltpu.make_async_copy(tab.at[pl.ds(0, bpw)], rows_v, s).wait()
        pltpu.sync_copy(rows_v, out.at[pl.ds(base, bpw)])


def _sc_gather(U2, M2, uidx, midx, batch):
    bpw = batch // NW
    mesh = plsc.VectorSubcoreMesh(core_axis_name="c", subcore_axis_name="s")
    f = pl.kernel(
        functools.partial(_gather_body, bpw),
        out_type=(jax.ShapeDtypeStruct((batch, 128), jnp.float32),
                  jax.ShapeDtypeStruct((batch, 128), jnp.float32)),
        mesh=mesh,
        scratch_types=[
            pltpu.VMEM((bpw,), jnp.int32),
            pltpu.VMEM((bpw,), jnp.int32),
            pltpu.VMEM((bpw, 128), jnp.float32),
            pltpu.SemaphoreType.DMA,
            pltpu.SemaphoreType.DMA,
        ],
    )
    return f(U2, M2, uidx, midx)


def _mlp_body(uw_ref, mw_ref, us_ref, ms_ref, w1a_ref, w1b_ref, b1_ref,
              w2_ref, b2_ref, w3_ref, b3_ref, o_ref):
    u_sel = (us_ref[...] & 1) == 0
    m_sel = (ms_ref[...] & 1) == 0
    uw = uw_ref[...]
    mw = mw_ref[...]
    u_emb = jnp.where(u_sel, uw[:, :64], uw[:, 64:])
    m_emb = jnp.where(m_sel, mw[:, :64], mw[:, 64:])
    h = (jnp.dot(u_emb, w1a_ref[...], preferred_element_type=jnp.float32)
         + jnp.dot(m_emb, w1b_ref[...], preferred_element_type=jnp.float32)
         + b1_ref[...])
    h = jnp.maximum(h, 0.0)
    h = jnp.dot(h, w2_ref[...], preferred_element_type=jnp.float32) + b2_ref[...]
    h = jnp.maximum(h, 0.0)
    z = jnp.sum(h * w3_ref[...], axis=1, keepdims=True) + b3_ref[...]
    o = 1.0 / (1.0 + jnp.exp(-z))
    o_ref[...] = o * 4.0 + 1.0


def _tc_mlp(u_wide, m_wide, users2d, movies2d, w1a, w1b, b1, w2, b2, w3, b3,
            batch, blk):
    grid = (batch // blk,)
    full = lambda i: (0, 0)
    return pl.pallas_call(
        _mlp_body,
        grid=grid,
        in_specs=[
            pl.BlockSpec((blk, 128), lambda i: (i, 0)),
            pl.BlockSpec((blk, 128), lambda i: (i, 0)),
            pl.BlockSpec((blk, 1), lambda i: (i, 0)),
            pl.BlockSpec((blk, 1), lambda i: (i, 0)),
            pl.BlockSpec((64, 128), full),
            pl.BlockSpec((64, 128), full),
            pl.BlockSpec((1, 128), full),
            pl.BlockSpec((128, 128), full),
            pl.BlockSpec((1, 128), full),
            pl.BlockSpec((1, 128), full),
            pl.BlockSpec((1, 1), full),
        ],
        out_specs=pl.BlockSpec((blk, 1), lambda i: (i, 0)),
        out_shape=jax.ShapeDtypeStruct((batch, 1), jnp.float32),
    )(u_wide, m_wide, users2d, movies2d, w1a, w1b, b1, w2, b2, w3, b3)


def kernel(users, movies, U, M, W1, b1, W2, b2, W3, b3):
    batch = users.shape[0]
    nf = U.shape[1]
    users = users.astype(jnp.int32)
    movies = movies.astype(jnp.int32)
    U2 = U.reshape(U.shape[0] // 2, 2 * nf)
    M2 = M.reshape(M.shape[0] // 2, 2 * nf)
    u_wide, m_wide = _sc_gather(U2, M2, users >> 1, movies >> 1, batch)
    w1a = W1[:, :nf].T
    w1b = W1[:, nf:].T
    out = _tc_mlp(u_wide, m_wide, users.reshape(-1, 1), movies.reshape(-1, 1),
                  w1a, w1b, b1.reshape(1, -1), W2.T, b2.reshape(1, -1),
                  W3, b3.reshape(1, 1), batch, 2048)
    return out


# speedup vs baseline: 1.9816x; 1.9816x over previous
import functools

import jax
import jax.numpy as jnp
from jax import lax
from jax.experimental import pallas as pl
from jax.experimental.pallas import tpu as pltpu
from jax.experimental.pallas import tpu_sc as plsc

NC = 2
NS = 16
NW = NC * NS
CHUNK = 32
BLK = 4096
TB = 32

_DN = (((0,), (0,)), ((), ()))


def _tconv_main_body(x1_ref, x2_ref, i64_ref, o_ref):
    i64 = i64_ref[...]
    o_ref[:, :64] = lax.dot_general(x1_ref[...], i64, _DN,
                                    preferred_element_type=jnp.float32)
    o_ref[:, 64:] = lax.dot_general(x2_ref[...], i64, _DN,
                                    preferred_element_type=jnp.float32)


def _tconv_tail_body(tail, tt, mo_ref, x_ref, i64_ref, o_ref):
    del mo_ref
    o_ref[...] = jnp.zeros((tt, 128), jnp.float32)
    o_ref[:tail, :64] = lax.dot_general(x_ref[...], i64_ref[...], _DN,
                                        preferred_element_type=jnp.float32)


def _tconv(V, i64):
    n = V.shape[1]
    nb = n // (2 * BLK)
    main_cols = nb * 2 * BLK
    main_rows = nb * BLK
    tail = n - main_cols
    tt = 1
    while tt < tail:
        tt *= 2
    rows = main_rows + (tt if tail else 0)
    full = lambda i: (0, 0)
    main = pl.pallas_call(
        _tconv_main_body,
        grid=(nb,),
        in_specs=[
            pl.BlockSpec((64, BLK), lambda i: (0, 2 * i)),
            pl.BlockSpec((64, BLK), lambda i: (0, 2 * i + 1)),
            pl.BlockSpec((64, 64), full),
        ],
        out_specs=pl.BlockSpec((BLK, 128), lambda i: (i, 0)),
        out_shape=jax.ShapeDtypeStruct((rows, 128), jnp.float32),
    )(V, V, i64)
    if tail == 0:
        return main
    vt = V[:, main_cols:]
    rb0 = main_rows // tt
    return pl.pallas_call(
        functools.partial(_tconv_tail_body, tail, tt),
        grid=(1,),
        in_specs=[
            pl.BlockSpec(memory_space=pl.ANY),
            pl.BlockSpec((64, tail), lambda i: (0, 0)),
            pl.BlockSpec((64, 64), full),
        ],
        out_specs=pl.BlockSpec((tt, 128), lambda i: (rb0, 0)),
        out_shape=jax.ShapeDtypeStruct((rows, 128), jnp.float32),
        input_output_aliases={0: 0},
    )(main, vt, i64)


def _route(idx, n):
    nb = n // (2 * BLK)
    main_cols = nb * 2 * BLK
    main_rows = nb * BLK
    j = idx // BLK
    in_main = idx < main_cols
    row = jnp.where(in_main, (j >> 1) * BLK + idx % BLK,
                    main_rows + (idx - main_cols))
    sel = jnp.where(in_main, j & 1, 0)
    return row, sel


def _gather_body(bpw, u_tab, m_tab, uidx_hbm, midx_hbm, u_out, m_out,
                 uidx_v, midx_v, rows_v, sem, sem2):
    wid = lax.axis_index("s") * NC + lax.axis_index("c")
    base = wid * bpw
    nchunk = bpw // CHUNK
    pltpu.sync_copy(uidx_hbm.at[pl.ds(base, bpw)], uidx_v)
    pltpu.sync_copy(midx_hbm.at[pl.ds(base, bpw)], midx_v)
    for idx_v, tab, out, s in ((uidx_v, u_tab, u_out, sem),
                               (midx_v, m_tab, m_out, sem2)):
        for j in range(nchunk):
            pltpu.async_copy(tab.at[idx_v.at[pl.ds(j * CHUNK, CHUNK)]],
                             rows_v.at[pl.ds(j * CHUNK, CHUNK)], s)
        pltpu.make_async_copy(tab.at[pl.ds(0, bpw)], rows_v, s).wait()
        pltpu.sync_copy(rows_v, out.at[pl.ds(base, bpw)])


def _sc_gather(U2, M2, uidx, midx, batch):
    bpw = batch // NW
    mesh = plsc.VectorSubcoreMesh(core_axis_name="c", subcore_axis_name="s")
    f = pl.kernel(
        functools.partial(_gather_body, bpw),
        out_type=(jax.ShapeDtypeStruct((batch, 128), jnp.float32),
                  jax.ShapeDtypeStruct((batch, 128), jnp.float32)),
        mesh=mesh,
        scratch_types=[
            pltpu.VMEM((bpw,), jnp.int32),
            pltpu.VMEM((bpw,), jnp.int32),
            pltpu.VMEM((bpw, 128), jnp.float32),
            pltpu.SemaphoreType.DMA,
            pltpu.SemaphoreType.DMA,
        ],
    )
    return f(U2, M2, uidx, midx)


def _mlp_body(uw_ref, mw_ref, us_ref, ms_ref, w1a_ref, w1b_ref, b1_ref,
              w2_ref, b2_ref, w3_ref, b3_ref, o_ref):
    u_sel = us_ref[...] == 0
    m_sel = ms_ref[...] == 0
    uw = uw_ref[...]
    mw = mw_ref[...]
    u_emb = jnp.where(u_sel, uw[:, :64], uw[:, 64:])
    m_emb = jnp.where(m_sel, mw[:, :64], mw[:, 64:])
    h = (jnp.dot(u_emb, w1a_ref[...], preferred_element_type=jnp.float32)
         + jnp.dot(m_emb, w1b_ref[...], preferred_element_type=jnp.float32)
         + b1_ref[...])
    h = jnp.maximum(h, 0.0)
    h = jnp.dot(h, w2_ref[...], preferred_element_type=jnp.float32) + b2_ref[...]
    h = jnp.maximum(h, 0.0)
    z = jnp.sum(h * w3_ref[...], axis=1, keepdims=True) + b3_ref[...]
    o = 1.0 / (1.0 + jnp.exp(-z))
    o_ref[...] = o * 4.0 + 1.0


def _tc_mlp(u_wide, m_wide, usel2d, msel2d, w1a, w1b, b1, w2, b2,
            w3, b3, batch, blk):
    grid = (batch // blk,)
    full = lambda i: (0, 0)
    return pl.pallas_call(
        _mlp_body,
        grid=grid,
        in_specs=[
            pl.BlockSpec((blk, 128), lambda i: (i, 0)),
            pl.BlockSpec((blk, 128), lambda i: (i, 0)),
            pl.BlockSpec((blk, 1), lambda i: (i, 0)),
            pl.BlockSpec((blk, 1), lambda i: (i, 0)),
            pl.BlockSpec((64, 128), full),
            pl.BlockSpec((64, 128), full),
            pl.BlockSpec((1, 128), full),
            pl.BlockSpec((128, 128), full),
            pl.BlockSpec((1, 128), full),
            pl.BlockSpec((1, 128), full),
            pl.BlockSpec((1, 1), full),
        ],
        out_specs=pl.BlockSpec((blk, 1), lambda i: (i, 0)),
        out_shape=jax.ShapeDtypeStruct((batch, 1), jnp.float32),
    )(u_wide, m_wide, usel2d, msel2d, w1a, w1b, b1, w2, b2, w3, b3)


def kernel(users, movies, U, M, W1, b1, W2, b2, W3, b3):
    batch = users.shape[0]
    nf = U.shape[1]
    users = users.astype(jnp.int32)
    movies = movies.astype(jnp.int32)
    i64 = jnp.eye(64, dtype=jnp.float32)
    U2 = _tconv(U.T, i64)
    M2 = _tconv(M.T, i64)
    uidx, usel = _route(users, U.shape[0])
    midx, msel = _route(movies, M.shape[0])
    u_wide, m_wide = _sc_gather(U2, M2, uidx, midx, batch)
    w1a = W1[:, :nf].T
    w1b = W1[:, nf:].T
    out = _tc_mlp(u_wide, m_wide, usel.reshape(-1, 1), msel.reshape(-1, 1),
                  w1a, w1b, b1.reshape(1, -1), W2.T, b2.reshape(1, -1),
                  W3, b3.reshape(1, 1), batch, 2048)
    return out


# speedup vs baseline: 2.2030x; 1.1118x over previous
import functools

import jax
import jax.numpy as jnp
from jax import lax
from jax.experimental import pallas as pl
from jax.experimental.pallas import tpu as pltpu
from jax.experimental.pallas import tpu_sc as plsc

NC = 2
NS = 16
NW = NC * NS
CHUNK = 32
BLK = 8192
TB = 32

_DN = (((0,), (0,)), ((), ()))


def _tconv_main_body(x1_ref, x2_ref, i64_ref, o_ref):
    del i64_ref
    o_ref[:, :64] = jnp.transpose(x1_ref[...])
    o_ref[:, 64:] = jnp.transpose(x2_ref[...])


def _tconv_tail_body(tail, tt, mo_ref, x_ref, i64_ref, o_ref):
    del mo_ref
    o_ref[...] = jnp.zeros((tt, 128), jnp.float32)
    o_ref[:tail, :64] = lax.dot_general(x_ref[...], i64_ref[...], _DN,
                                        preferred_element_type=jnp.float32)


def _tconv(V, i64):
    n = V.shape[1]
    nb = n // (2 * BLK)
    main_cols = nb * 2 * BLK
    main_rows = nb * BLK
    tail = n - main_cols
    tt = 1
    while tt < tail:
        tt *= 2
    rows = main_rows + (tt if tail else 0)
    full = lambda i: (0, 0)
    main = pl.pallas_call(
        _tconv_main_body,
        grid=(nb,),
        in_specs=[
            pl.BlockSpec((64, BLK), lambda i: (0, 2 * i)),
            pl.BlockSpec((64, BLK), lambda i: (0, 2 * i + 1)),
            pl.BlockSpec((64, 64), full),
        ],
        out_specs=pl.BlockSpec((BLK, 128), lambda i: (i, 0)),
        out_shape=jax.ShapeDtypeStruct((rows, 128), jnp.float32),
    )(V, V, i64)
    if tail == 0:
        return main
    vt = V[:, main_cols:]
    rb0 = main_rows // tt
    return pl.pallas_call(
        functools.partial(_tconv_tail_body, tail, tt),
        grid=(1,),
        in_specs=[
            pl.BlockSpec(memory_space=pl.ANY),
            pl.BlockSpec((64, tail), lambda i: (0, 0)),
            pl.BlockSpec((64, 64), full),
        ],
        out_specs=pl.BlockSpec((tt, 128), lambda i: (rb0, 0)),
        out_shape=jax.ShapeDtypeStruct((rows, 128), jnp.float32),
        input_output_aliases={0: 0},
    )(main, vt, i64)


def _route(idx, n):
    nb = n // (2 * BLK)
    main_cols = nb * 2 * BLK
    main_rows = nb * BLK
    j = idx // BLK
    in_main = idx < main_cols
    row = jnp.where(in_main, (j >> 1) * BLK + idx % BLK,
                    main_rows + (idx - main_cols))
    sel = jnp.where(in_main, j & 1, 0)
    return row, sel


def _gather_body(bpw, u_tab, m_tab, uidx_hbm, midx_hbm, u_out, m_out,
                 uidx_v, midx_v, rows_v, sem, sem2):
    wid = lax.axis_index("s") * NC + lax.axis_index("c")
    base = wid * bpw
    nchunk = bpw // CHUNK
    pltpu.sync_copy(uidx_hbm.at[pl.ds(base, bpw)], uidx_v)
    pltpu.sync_copy(midx_hbm.at[pl.ds(base, bpw)], midx_v)
    for idx_v, tab, out, s in ((uidx_v, u_tab, u_out, sem),
                               (midx_v, m_tab, m_out, sem2)):
        for j in range(nchunk):
            pltpu.async_copy(tab.at[idx_v.at[pl.ds(j * CHUNK, CHUNK)]],
                             rows_v.at[pl.ds(j * CHUNK, CHUNK)], s)
        pltpu.make_async_copy(tab.at[pl.ds(0, bpw)], rows_v, s).wait()
        pltpu.sync_copy(rows_v, out.at[pl.ds(base, bpw)])


def _sc_gather(U2, M2, uidx, midx, batch):
    bpw = batch // NW
    mesh = plsc.VectorSubcoreMesh(core_axis_name="c", subcore_axis_name="s")
    f = pl.kernel(
        functools.partial(_gather_body, bpw),
        out_type=(jax.ShapeDtypeStruct((batch, 128), jnp.float32),
                  jax.ShapeDtypeStruct((batch, 128), jnp.float32)),
        mesh=mesh,
        scratch_types=[
            pltpu.VMEM((bpw,), jnp.int32),
            pltpu.VMEM((bpw,), jnp.int32),
            pltpu.VMEM((bpw, 128), jnp.float32),
            pltpu.SemaphoreType.DMA,
            pltpu.SemaphoreType.DMA,
        ],
    )
    return f(U2, M2, uidx, midx)


def _mlp_body(uw_ref, mw_ref, us_ref, ms_ref, w1a_ref, w1b_ref, b1_ref,
              w2_ref, b2_ref, w3_ref, b3_ref, o_ref):
    u_sel = us_ref[...] == 0
    m_sel = ms_ref[...] == 0
    uw = uw_ref[...]
    mw = mw_ref[...]
    u_emb = jnp.where(u_sel, uw[:, :64], uw[:, 64:])
    m_emb = jnp.where(m_sel, mw[:, :64], mw[:, 64:])
    h = (jnp.dot(u_emb, w1a_ref[...], preferred_element_type=jnp.float32)
         + jnp.dot(m_emb, w1b_ref[...], preferred_element_type=jnp.float32)
         + b1_ref[...])
    h = jnp.maximum(h, 0.0)
    h = jnp.dot(h, w2_ref[...], preferred_element_type=jnp.float32) + b2_ref[...]
    h = jnp.maximum(h, 0.0)
    z = jnp.sum(h * w3_ref[...], axis=1, keepdims=True) + b3_ref[...]
    o = 1.0 / (1.0 + jnp.exp(-z))
    o_ref[...] = o * 4.0 + 1.0


def _tc_mlp(u_wide, m_wide, usel2d, msel2d, w1a, w1b, b1, w2, b2,
            w3, b3, batch, blk):
    grid = (batch // blk,)
    full = lambda i: (0, 0)
    return pl.pallas_call(
        _mlp_body,
        grid=grid,
        in_specs=[
            pl.BlockSpec((blk, 128), lambda i: (i, 0)),
            pl.BlockSpec((blk, 128), lambda i: (i, 0)),
            pl.BlockSpec((blk, 1), lambda i: (i, 0)),
            pl.BlockSpec((blk, 1), lambda i: (i, 0)),
            pl.BlockSpec((64, 128), full),
            pl.BlockSpec((64, 128), full),
            pl.BlockSpec((1, 128), full),
            pl.BlockSpec((128, 128), full),
            pl.BlockSpec((1, 128), full),
            pl.BlockSpec((1, 128), full),
            pl.BlockSpec((1, 1), full),
        ],
        out_specs=pl.BlockSpec((blk, 1), lambda i: (i, 0)),
        out_shape=jax.ShapeDtypeStruct((batch, 1), jnp.float32),
    )(u_wide, m_wide, usel2d, msel2d, w1a, w1b, b1, w2, b2, w3, b3)


def kernel(users, movies, U, M, W1, b1, W2, b2, W3, b3):
    batch = users.shape[0]
    nf = U.shape[1]
    users = users.astype(jnp.int32)
    movies = movies.astype(jnp.int32)
    i64 = jnp.eye(64, dtype=jnp.float32)
    U2 = _tconv(U.T, i64)
    M2 = _tconv(M.T, i64)
    uidx, usel = _route(users, U.shape[0])
    midx, msel = _route(movies, M.shape[0])
    u_wide, m_wide = _sc_gather(U2, M2, uidx, midx, batch)
    w1a = W1[:, :nf].T
    w1b = W1[:, nf:].T
    out = _tc_mlp(u_wide, m_wide, usel.reshape(-1, 1), msel.reshape(-1, 1),
                  w1a, w1b, b1.reshape(1, -1), W2.T, b2.reshape(1, -1),
                  W3, b3.reshape(1, 1), batch, 2048)
    return out


# speedup vs baseline: 2.3308x; 1.0580x over previous
import functools

import jax
import jax.numpy as jnp
from jax import lax
from jax.experimental import pallas as pl
from jax.experimental.pallas import tpu as pltpu
from jax.experimental.pallas import tpu_sc as plsc

NC = 2
NS = 16
NW = NC * NS
CHUNK = 32
BLK = 8192
TB = 32

_DN = (((0,), (0,)), ((), ()))


def _tconv_main_body(x1_ref, x2_ref, i64_ref, o_ref):
    del i64_ref
    o_ref[:, :64] = jnp.transpose(x1_ref[...])
    o_ref[:, 64:] = jnp.transpose(x2_ref[...])


def _tconv_tail_body(tail, tt, mo_ref, x_ref, i64_ref, o_ref):
    del mo_ref
    o_ref[...] = jnp.zeros((tt, 128), jnp.float32)
    o_ref[:tail, :64] = lax.dot_general(x_ref[...], i64_ref[...], _DN,
                                        preferred_element_type=jnp.float32)


def _tconv(V, i64):
    n = V.shape[1]
    nb = n // (2 * BLK)
    main_cols = nb * 2 * BLK
    main_rows = nb * BLK
    tail = n - main_cols
    tt = 1
    while tt < tail:
        tt *= 2
    rows = main_rows + (tt if tail else 0)
    full = lambda i: (0, 0)
    main = pl.pallas_call(
        _tconv_main_body,
        grid=(nb,),
        in_specs=[
            pl.BlockSpec((64, BLK), lambda i: (0, 2 * i)),
            pl.BlockSpec((64, BLK), lambda i: (0, 2 * i + 1)),
            pl.BlockSpec((64, 64), full),
        ],
        out_specs=pl.BlockSpec((BLK, 128), lambda i: (i, 0)),
        out_shape=jax.ShapeDtypeStruct((rows, 128), jnp.float32),
    )(V, V, i64)
    if tail == 0:
        return main
    vt = V[:, main_cols:]
    rb0 = main_rows // tt
    return pl.pallas_call(
        functools.partial(_tconv_tail_body, tail, tt),
        grid=(1,),
        in_specs=[
            pl.BlockSpec(memory_space=pl.ANY),
            pl.BlockSpec((64, tail), lambda i: (0, 0)),
            pl.BlockSpec((64, 64), full),
        ],
        out_specs=pl.BlockSpec((tt, 128), lambda i: (rb0, 0)),
        out_shape=jax.ShapeDtypeStruct((rows, 128), jnp.float32),
        input_output_aliases={0: 0},
    )(main, vt, i64)


def _route(idx, n):
    nb = n // (2 * BLK)
    main_cols = nb * 2 * BLK
    main_rows = nb * BLK
    j = idx // BLK
    in_main = idx < main_cols
    row = jnp.where(in_main, (j >> 1) * BLK + idx % BLK,
                    main_rows + (idx - main_cols))
    sel = jnp.where(in_main, j & 1, 0)
    return row, sel


def _gather_body(bpw, tab, idx_hbm, out, idx_v, rows_v, sem):
    wid = lax.axis_index("s") * NC + lax.axis_index("c")
    base = wid * bpw
    nchunk = bpw // CHUNK
    pltpu.sync_copy(idx_hbm.at[pl.ds(base, bpw)], idx_v)
    for j in range(nchunk):
        pltpu.async_copy(tab.at[idx_v.at[pl.ds(j * CHUNK, CHUNK)]],
                         rows_v.at[pl.ds(j * CHUNK, CHUNK)], sem)
    pltpu.make_async_copy(tab.at[pl.ds(0, bpw)], rows_v, sem).wait()
    pltpu.sync_copy(rows_v, out.at[pl.ds(base, bpw)])


def _sc_gather(tab, idx, batch):
    bpw = batch // NW
    mesh = plsc.VectorSubcoreMesh(core_axis_name="c", subcore_axis_name="s")
    f = pl.kernel(
        functools.partial(_gather_body, bpw),
        out_type=jax.ShapeDtypeStruct((batch, 128), jnp.float32),
        mesh=mesh,
        scratch_types=[
            pltpu.VMEM((bpw,), jnp.int32),
            pltpu.VMEM((bpw, 128), jnp.float32),
            pltpu.SemaphoreType.DMA,
        ],
    )
    return f(tab, idx)


_DN1 = (((1,), (1,)), ((), ()))


def _mlp_body(uw_ref, mw_ref, us_ref, ms_ref, w1_ref, b1_ref,
              w2_ref, b2_ref, w3_ref, b3_ref, o_ref):
    u_sel = us_ref[...] == 0
    m_sel = ms_ref[...] == 0
    uw = uw_ref[...]
    mw = mw_ref[...]
    u_emb = jnp.where(u_sel, uw[:, :64], uw[:, 64:])
    m_emb = jnp.where(m_sel, mw[:, :64], mw[:, 64:])
    w1 = w1_ref[...]
    h = (lax.dot_general(u_emb, w1[:, :64], _DN1,
                         preferred_element_type=jnp.float32)
         + lax.dot_general(m_emb, w1[:, 64:], _DN1,
                           preferred_element_type=jnp.float32)
         + b1_ref[...])
    h = jnp.maximum(h, 0.0)
    h = lax.dot_general(h, w2_ref[...], _DN1,
                        preferred_element_type=jnp.float32) + b2_ref[...]
    h = jnp.maximum(h, 0.0)
    z = lax.dot_general(w3_ref[...], h, _DN1,
                        preferred_element_type=jnp.float32) + b3_ref[...]
    o = 1.0 / (1.0 + jnp.exp(-z))
    o_ref[...] = o * 4.0 + 1.0


def _tc_mlp(u_wide, m_wide, usel2d, msel2d, w1, b1, w2, b2, w3, b3,
            batch, blk):
    grid = (batch // blk,)
    full = lambda i: (0, 0)
    return pl.pallas_call(
        _mlp_body,
        grid=grid,
        in_specs=[
            pl.BlockSpec((blk, 128), lambda i: (i, 0)),
            pl.BlockSpec((blk, 128), lambda i: (i, 0)),
            pl.BlockSpec((blk, 1), lambda i: (i, 0)),
            pl.BlockSpec((blk, 1), lambda i: (i, 0)),
            pl.BlockSpec((128, 128), full),
            pl.BlockSpec((1, 128), full),
            pl.BlockSpec((128, 128), full),
            pl.BlockSpec((1, 128), full),
            pl.BlockSpec((1, 128), full),
            pl.BlockSpec((1, 1), full),
        ],
        out_specs=pl.BlockSpec((1, blk), lambda i: (0, i)),
        out_shape=jax.ShapeDtypeStruct((1, batch), jnp.float32),
    )(u_wide, m_wide, usel2d, msel2d, w1, b1, w2, b2, w3, b3)


def kernel(users, movies, U, M, W1, b1, W2, b2, W3, b3):
    batch = users.shape[0]
    users = users.astype(jnp.int32)
    movies = movies.astype(jnp.int32)
    i64 = jnp.eye(64, dtype=jnp.float32)
    M2 = _tconv(M.T, i64)
    midx, msel = _route(movies, M.shape[0])
    m_wide = _sc_gather(M2, midx, batch)
    U2 = _tconv(U.T, i64)
    uidx, usel = _route(users, U.shape[0])
    u_wide = _sc_gather(U2, uidx, batch)
    out = _tc_mlp(u_wide, m_wide,
                  usel.astype(jnp.int8).reshape(-1, 1),
                  msel.astype(jnp.int8).reshape(-1, 1),
                  W1, b1.reshape(1, -1), W2, b2.reshape(1, -1),
                  W3, b3.reshape(1, 1), batch, 2048)
    return out.T


# speedup vs baseline: 2.3748x; 1.0189x over previous
import functools

import jax
import jax.numpy as jnp
from jax import lax
from jax.experimental import pallas as pl
from jax.experimental.pallas import tpu as pltpu
from jax.experimental.pallas import tpu_sc as plsc

NC = 2
NS = 16
NW = NC * NS
CHUNK = 32
BLK = 16384
TB = 32

_DN = (((0,), (0,)), ((), ()))


def _tconv_main_body(x1_ref, x2_ref, i64_ref, o_ref):
    del i64_ref
    o_ref[:, :64] = jnp.transpose(x1_ref[...])
    o_ref[:, 64:] = jnp.transpose(x2_ref[...])


def _tconv_tail_body(tail, tt, mo_ref, x_ref, i64_ref, o_ref):
    del mo_ref
    o_ref[...] = jnp.zeros((tt, 128), jnp.float32)
    o_ref[:tail, :64] = lax.dot_general(x_ref[...], i64_ref[...], _DN,
                                        preferred_element_type=jnp.float32)


def _tconv(V, i64):
    n = V.shape[1]
    nb = n // (2 * BLK)
    main_cols = nb * 2 * BLK
    main_rows = nb * BLK
    tail = n - main_cols
    tt = 1
    while tt < tail:
        tt *= 2
    rows = main_rows + (tt if tail else 0)
    full = lambda i: (0, 0)
    main = pl.pallas_call(
        _tconv_main_body,
        grid=(nb,),
        in_specs=[
            pl.BlockSpec((64, BLK), lambda i: (0, 2 * i)),
            pl.BlockSpec((64, BLK), lambda i: (0, 2 * i + 1)),
            pl.BlockSpec((64, 64), full),
        ],
        out_specs=pl.BlockSpec((BLK, 128), lambda i: (i, 0)),
        out_shape=jax.ShapeDtypeStruct((rows, 128), jnp.float32),
    )(V, V, i64)
    if tail == 0:
        return main
    vt = V[:, main_cols:]
    rb0 = main_rows // tt
    return pl.pallas_call(
        functools.partial(_tconv_tail_body, tail, tt),
        grid=(1,),
        in_specs=[
            pl.BlockSpec(memory_space=pl.ANY),
            pl.BlockSpec((64, tail), lambda i: (0, 0)),
            pl.BlockSpec((64, 64), full),
        ],
        out_specs=pl.BlockSpec((tt, 128), lambda i: (rb0, 0)),
        out_shape=jax.ShapeDtypeStruct((rows, 128), jnp.float32),
        input_output_aliases={0: 0},
    )(main, vt, i64)


def _route(idx, n):
    nb = n // (2 * BLK)
    main_cols = nb * 2 * BLK
    main_rows = nb * BLK
    j = idx // BLK
    in_main = idx < main_cols
    row = jnp.where(in_main, (j >> 1) * BLK + idx % BLK,
                    main_rows + (idx - main_cols))
    sel = jnp.where(in_main, j & 1, 0)
    return row, sel


def _gather_body(bpw, tab, idx_hbm, out, idx_v, rows_v, sem):
    wid = lax.axis_index("s") * NC + lax.axis_index("c")
    base = wid * bpw
    nchunk = bpw // CHUNK
    pltpu.sync_copy(idx_hbm.at[pl.ds(base, bpw)], idx_v)
    for j in range(nchunk):
        pltpu.async_copy(tab.at[idx_v.at[pl.ds(j * CHUNK, CHUNK)]],
                         rows_v.at[pl.ds(j * CHUNK, CHUNK)], sem)
    pltpu.make_async_copy(tab.at[pl.ds(0, bpw)], rows_v, sem).wait()
    pltpu.sync_copy(rows_v, out.at[pl.ds(base, bpw)])


def _sc_gather(tab, idx, batch):
    bpw = batch // NW
    mesh = plsc.VectorSubcoreMesh(core_axis_name="c", subcore_axis_name="s")
    f = pl.kernel(
        functools.partial(_gather_body, bpw),
        out_type=jax.ShapeDtypeStruct((batch, 128), jnp.float32),
        mesh=mesh,
        scratch_types=[
            pltpu.VMEM((bpw,), jnp.int32),
            pltpu.VMEM((bpw, 128), jnp.float32),
            pltpu.SemaphoreType.DMA,
        ],
    )
    return f(tab, idx)


_DN1 = (((1,), (1,)), ((), ()))


def _mlp_body(uw_ref, mw_ref, us_ref, ms_ref, w1_ref, b1_ref,
              w2_ref, b2_ref, w3_ref, b3_ref, o_ref):
    u_sel = us_ref[...] == 0
    m_sel = ms_ref[...] == 0
    uw = uw_ref[...]
    mw = mw_ref[...]
    u_emb = jnp.where(u_sel, uw[:, :64], uw[:, 64:])
    m_emb = jnp.where(m_sel, mw[:, :64], mw[:, 64:])
    w1 = w1_ref[...]
    h = (lax.dot_general(u_emb, w1[:, :64], _DN1,
                         preferred_element_type=jnp.float32)
         + lax.dot_general(m_emb, w1[:, 64:], _DN1,
                           preferred_element_type=jnp.float32)
         + b1_ref[...])
    h = jnp.maximum(h, 0.0)
    h = lax.dot_general(h, w2_ref[...], _DN1,
                        preferred_element_type=jnp.float32) + b2_ref[...]
    h = jnp.maximum(h, 0.0)
    z = lax.dot_general(w3_ref[...], h, _DN1,
                        preferred_element_type=jnp.float32) + b3_ref[...]
    o = 1.0 / (1.0 + jnp.exp(-z))
    o_ref[...] = o * 4.0 + 1.0


def _tc_mlp(u_wide, m_wide, usel2d, msel2d, w1, b1, w2, b2, w3, b3,
            batch, blk):
    grid = (batch // blk,)
    full = lambda i: (0, 0)
    return pl.pallas_call(
        _mlp_body,
        grid=grid,
        in_specs=[
            pl.BlockSpec((blk, 128), lambda i: (i, 0)),
            pl.BlockSpec((blk, 128), lambda i: (i, 0)),
            pl.BlockSpec((blk, 1), lambda i: (i, 0)),
            pl.BlockSpec((blk, 1), lambda i: (i, 0)),
            pl.BlockSpec((128, 128), full),
            pl.BlockSpec((1, 128), full),
            pl.BlockSpec((128, 128), full),
            pl.BlockSpec((1, 128), full),
            pl.BlockSpec((1, 128), full),
            pl.BlockSpec((1, 1), full),
        ],
        out_specs=pl.BlockSpec((1, blk), lambda i: (0, i)),
        out_shape=jax.ShapeDtypeStruct((1, batch), jnp.float32),
    )(u_wide, m_wide, usel2d, msel2d, w1, b1, w2, b2, w3, b3)


def kernel(users, movies, U, M, W1, b1, W2, b2, W3, b3):
    batch = users.shape[0]
    users = users.astype(jnp.int32)
    movies = movies.astype(jnp.int32)
    i64 = jnp.eye(64, dtype=jnp.float32)
    M2 = _tconv(M.T, i64)
    midx, msel = _route(movies, M.shape[0])
    m_wide = _sc_gather(M2, midx, batch)
    U2 = _tconv(U.T, i64)
    uidx, usel = _route(users, U.shape[0])
    u_wide = _sc_gather(U2, uidx, batch)
    out = _tc_mlp(u_wide, m_wide,
                  usel.astype(jnp.int8).reshape(-1, 1),
                  msel.astype(jnp.int8).reshape(-1, 1),
                  W1, b1.reshape(1, -1), W2, b2.reshape(1, -1),
                  W3, b3.reshape(1, 1), batch, 2048)
    return out.T
